# Initial kernel scaffold; baseline (speedup 1.0000x reference)
import functools

import jax
import jax.numpy as jnp
from jax import lax
from jax.experimental import pallas as pl
from jax.experimental.pallas import tpu as pltpu
from jax.experimental.pallas import tpu_sc as plsc

N = 10000
E = 320000
H = 128
H2 = 256
L = 7
G = 64

NSUB = 16
CH = 128
NCH_PER_TILE = 157
EPT = NCH_PER_TILE * CH
EPAD = EPT * NSUB
NPAD = 10016
ZR = NPAD // NSUB

f32 = jnp.float32


def _in_body(x_ref, w_ref, b_ref, o_ref):
    o_ref[...] = (
        jnp.dot(x_ref[...], w_ref[...], preferred_element_type=f32) + b_ref[...]
    )


def _input_proj(x, W_in, b_in):
    BM = 2000
    return pl.pallas_call(
        _in_body,
        grid=(N // BM,),
        in_specs=[
            pl.BlockSpec((BM, H), lambda i: (i, 0)),
            pl.BlockSpec((H, H), lambda i: (0, 0)),
            pl.BlockSpec((1, H), lambda i: (0, 0)),
        ],
        out_specs=pl.BlockSpec((BM, H), lambda i: (i, 0)),
        out_shape=jax.ShapeDtypeStruct((N, H), f32),
    )(x, W_in, b_in.reshape(1, H))


def _ln(h, scale, bias):
    mu = jnp.mean(h, axis=1, keepdims=True)
    var = jnp.mean((h - mu) ** 2, axis=1, keepdims=True)
    return (h - mu) / jnp.sqrt(var + 1e-5) * scale + bias


def _pre_body(h_ref, ls_ref, lb_ref, tv_ref, hn_ref, r_ref, q_ref):
    hn = jnp.maximum(_ln(h_ref[...], ls_ref[...], lb_ref[...]), 0.0)
    msg = hn + 1e-7
    q = jnp.exp(msg * tv_ref[...])
    hn_ref[...] = hn
    q_ref[...] = q
    r_ref[...] = msg * q


def _pre(h, ls, lb, tv):
    BM = 2000
    return pl.pallas_call(
        _pre_body,
        grid=(N // BM,),
        in_specs=[
            pl.BlockSpec((BM, H), lambda i: (i, 0)),
            pl.BlockSpec((1, H), lambda i: (0, 0)),
            pl.BlockSpec((1, H), lambda i: (0, 0)),
            pl.BlockSpec((1, H), lambda i: (0, 0)),
        ],
        out_specs=[pl.BlockSpec((BM, H), lambda i: (i, 0))] * 3,
        out_shape=[jax.ShapeDtypeStruct((N, H), f32)] * 3,
    )(h, ls, lb, tv)


def _post_body(h_ref, hn_ref, nu_ref, de_ref, w1_ref, b1_ref, mls_ref,
               mlb_ref, w2_ref, b2_ref, o_ref):
    aggr = nu_ref[...] / (de_ref[...] + 1e-16)
    out0 = aggr + hn_ref[...]
    u = jnp.dot(out0, w1_ref[...], preferred_element_type=f32) + b1_ref[...]
    u = jnp.maximum(_ln(u, mls_ref[...], mlb_ref[...]), 0.0)
    v = jnp.dot(u, w2_ref[...], preferred_element_type=f32) + b2_ref[...]
    o_ref[...] = h_ref[...] + v


def _post(h, hn, numer, denom, W1, b1, mls, mlb, W2, b2):
    BM = 2000
    return pl.pallas_call(
        _post_body,
        grid=(N // BM,),
        in_specs=[
            pl.BlockSpec((BM, H), lambda i: (i, 0)),
            pl.BlockSpec((BM, H), lambda i: (i, 0)),
            pl.BlockSpec((BM, H), lambda i: (i, 0)),
            pl.BlockSpec((BM, H), lambda i: (i, 0)),
            pl.BlockSpec((H, H2), lambda i: (0, 0)),
            pl.BlockSpec((1, H2), lambda i: (0, 0)),
            pl.BlockSpec((1, H2), lambda i: (0, 0)),
            pl.BlockSpec((1, H2), lambda i: (0, 0)),
            pl.BlockSpec((H2, H), lambda i: (0, 0)),
            pl.BlockSpec((1, H), lambda i: (0, 0)),
        ],
        out_specs=pl.BlockSpec((BM, H), lambda i: (i, 0)),
        out_shape=jax.ShapeDtypeStruct((N, H), f32),
    )(h, hn, numer, denom, W1, b1, mls, mlb, W2, b2)


def _head_body(h_ref, ls_ref, lb_ref, b_ref, wc1_ref, bc1_ref, wc2_ref,
               bc2_ref, wc3_ref, bc3_ref, o_ref):
    hn = jnp.maximum(_ln(h_ref[...], ls_ref[...], lb_ref[...]), 0.0)
    bvec = b_ref[0, :]
    gids = lax.broadcasted_iota(jnp.int32, (G, N), 0)
    oh = (gids == bvec[None, :]).astype(f32)
    summ = jnp.dot(oh, hn, preferred_element_type=f32)
    cnt = jnp.sum(oh, axis=1, keepdims=True)
    pooled = summ / jnp.maximum(cnt, 1.0)
    z = jnp.maximum(
        jnp.dot(pooled, wc1_ref[...], preferred_element_type=f32) + bc1_ref[...],
        0.0)
    z = jnp.maximum(
        jnp.dot(z, wc2_ref[...], preferred_element_type=f32) + bc2_ref[...],
        0.0)
    z3 = jnp.dot(z, wc3_ref[...], preferred_element_type=f32) + bc3_ref[...]
    o_ref[...] = z3


def _head(h, ls, lb, batch2d, Wc1, bc1, Wc2, bc2, Wc3b, bc3b):
    return pl.pallas_call(
        _head_body,
        out_shape=jax.ShapeDtypeStruct((G, H), f32),
    )(h, ls, lb, batch2d, Wc1, bc1, Wc2, bc2, Wc3b, bc3b)


def _edge_body(r_hbm, q_hbm, src_hbm, dst_hbm, zero_hbm, numer_hbm, denom_hbm,
               src_v, dst_v, rows_v, acc):
    c = lax.axis_index("c")
    s = lax.axis_index("s")

    pltpu.sync_copy(zero_hbm, acc.at[pl.ds(s * ZR, ZR)])
    plsc.subcore_barrier()

    base = s * EPT

    @pl.loop(0, NCH_PER_TILE)
    def _chunk(k):
        e0 = base + k * CH
        pltpu.sync_copy(src_hbm.at[pl.ds(e0, CH)], src_v)
        pltpu.sync_copy(dst_hbm.at[pl.ds(e0, CH)], dst_v)

        @pl.when(c == 0)
        def _():
            pltpu.sync_copy(r_hbm.at[src_v], rows_v)

        @pl.when(c == 1)
        def _():
            pltpu.sync_copy(q_hbm.at[src_v], rows_v)

        pltpu.sync_copy(rows_v, acc.at[dst_v], add=True)

    plsc.subcore_barrier()

    @pl.when(c == 0)
    def _():
        pltpu.sync_copy(acc.at[pl.ds(s * ZR, ZR)],
                        numer_hbm.at[pl.ds(s * ZR, ZR)])

    @pl.when(c == 1)
    def _():
        pltpu.sync_copy(acc.at[pl.ds(s * ZR, ZR)],
                        denom_hbm.at[pl.ds(s * ZR, ZR)])


@functools.lru_cache(maxsize=1)
def _edge_call_built():
    return pl.kernel(
        _edge_body,
        out_type=[jax.ShapeDtypeStruct((NPAD, H), f32),
                  jax.ShapeDtypeStruct((NPAD, H), f32)],
        mesh=plsc.VectorSubcoreMesh(core_axis_name="c", subcore_axis_name="s"),
        scratch_types=[
            pltpu.VMEM((CH,), jnp.int32),
            pltpu.VMEM((CH,), jnp.int32),
            pltpu.VMEM((CH, H), f32),
            pltpu.VMEM_SHARED((NPAD, H), f32),
        ],
    )


def _edge_call(r, q, srcp, dstp, zeros):
    return _edge_call_built()(r, q, srcp, dstp, zeros)


def kernel(x, edge_index, batch, W_in, b_in, ln_scale, ln_bias, t, W1, b1,
           mlp_ln_scale, mlp_ln_bias, W2, b2, Wc1, bc1, Wc2, bc2, Wc3, bc3):
    src = edge_index[0]
    dst = edge_index[1]
    pad = EPAD - E
    srcp = jnp.concatenate([src, jnp.zeros((pad,), jnp.int32)])
    dstp = jnp.concatenate([dst, jnp.full((pad,), NPAD - 8, jnp.int32)])
    zeros = jnp.zeros((ZR, H), f32)

    h = _input_proj(x, W_in, b_in)
    for i in range(L):
        tv = jnp.full((1, H), t[i], f32)
        hn, r, q = _pre(h, ln_scale[i].reshape(1, H), ln_bias[i].reshape(1, H),
                        tv)
        numer, denom = _edge_call(r, q, srcp, dstp, zeros)
        h = _post(h, hn, numer[:N], denom[:N], W1[i], b1[i].reshape(1, H2),
                  mlp_ln_scale[i].reshape(1, H2), mlp_ln_bias[i].reshape(1, H2),
                  W2[i], b2[i].reshape(1, H))

    out2d = _head(
        h, ln_scale[0].reshape(1, H), ln_bias[0].reshape(1, H),
        batch.reshape(1, N), Wc1, bc1.reshape(1, H), Wc2,
        bc2.reshape(1, G), jnp.tile(Wc3, (1, H)),
        jnp.broadcast_to(bc3.reshape(1, 1), (1, H)))
    return out2d[:, 0]


# speedup vs baseline: 8.3190x; 8.3190x over previous
import functools

import jax
import jax.numpy as jnp
from jax import lax
from jax.experimental import pallas as pl
from jax.experimental.pallas import tpu as pltpu
from jax.experimental.pallas import tpu_sc as plsc

N = 10000
E = 320000
H = 128
H2 = 256
L = 7
G = 64

NSUB = 16
CH = 128
NCH_PER_TILE = 157
EPT = NCH_PER_TILE * CH
EPAD = EPT * NSUB
NPAD = 10112
ZR = NPAD // NSUB

f32 = jnp.float32


def _in_body(x_ref, w_ref, b_ref, o_ref):
    o_ref[...] = (
        jnp.dot(x_ref[...], w_ref[...], preferred_element_type=f32) + b_ref[...]
    )


def _input_proj(x, W_in, b_in):
    BM = 2000
    return pl.pallas_call(
        _in_body,
        grid=(N // BM,),
        in_specs=[
            pl.BlockSpec((BM, H), lambda i: (i, 0)),
            pl.BlockSpec((H, H), lambda i: (0, 0)),
            pl.BlockSpec((1, H), lambda i: (0, 0)),
        ],
        out_specs=pl.BlockSpec((BM, H), lambda i: (i, 0)),
        out_shape=jax.ShapeDtypeStruct((N, H), f32),
    )(x, W_in, b_in.reshape(1, H))


def _ln(h, scale, bias):
    mu = jnp.mean(h, axis=1, keepdims=True)
    var = jnp.mean((h - mu) ** 2, axis=1, keepdims=True)
    return (h - mu) / jnp.sqrt(var + 1e-5) * scale + bias


def _pre_body(h_ref, ls_ref, lb_ref, tv_ref, hn_ref, r_ref, q_ref):
    hn = jnp.maximum(_ln(h_ref[...], ls_ref[...], lb_ref[...]), 0.0)
    msg = hn + 1e-7
    q = jnp.exp(msg * tv_ref[...])
    hn_ref[...] = hn
    q_ref[...] = q
    r_ref[...] = msg * q


def _pre(h, ls, lb, tv):
    BM = 2000
    return pl.pallas_call(
        _pre_body,
        grid=(N // BM,),
        in_specs=[
            pl.BlockSpec((BM, H), lambda i: (i, 0)),
            pl.BlockSpec((1, H), lambda i: (0, 0)),
            pl.BlockSpec((1, H), lambda i: (0, 0)),
            pl.BlockSpec((1, H), lambda i: (0, 0)),
        ],
        out_specs=[pl.BlockSpec((BM, H), lambda i: (i, 0))] * 3,
        out_shape=[jax.ShapeDtypeStruct((N, H), f32)] * 3,
    )(h, ls, lb, tv)


def _post_body(h_ref, hn_ref, nu_ref, de_ref, w1_ref, b1_ref, mls_ref,
               mlb_ref, w2_ref, b2_ref, o_ref):
    aggr = nu_ref[...] / (de_ref[...] + 1e-16)
    out0 = aggr + hn_ref[...]
    u = jnp.dot(out0, w1_ref[...], preferred_element_type=f32) + b1_ref[...]
    u = jnp.maximum(_ln(u, mls_ref[...], mlb_ref[...]), 0.0)
    v = jnp.dot(u, w2_ref[...], preferred_element_type=f32) + b2_ref[...]
    o_ref[...] = h_ref[...] + v


def _post(h, hn, numer, denom, W1, b1, mls, mlb, W2, b2):
    BM = 2000
    return pl.pallas_call(
        _post_body,
        grid=(N // BM,),
        in_specs=[
            pl.BlockSpec((BM, H), lambda i: (i, 0)),
            pl.BlockSpec((BM, H), lambda i: (i, 0)),
            pl.BlockSpec((BM, H), lambda i: (i, 0)),
            pl.BlockSpec((BM, H), lambda i: (i, 0)),
            pl.BlockSpec((H, H2), lambda i: (0, 0)),
            pl.BlockSpec((1, H2), lambda i: (0, 0)),
            pl.BlockSpec((1, H2), lambda i: (0, 0)),
            pl.BlockSpec((1, H2), lambda i: (0, 0)),
            pl.BlockSpec((H2, H), lambda i: (0, 0)),
            pl.BlockSpec((1, H), lambda i: (0, 0)),
        ],
        out_specs=pl.BlockSpec((BM, H), lambda i: (i, 0)),
        out_shape=jax.ShapeDtypeStruct((N, H), f32),
    )(h, hn, numer, denom, W1, b1, mls, mlb, W2, b2)


def _head_body(h_ref, ls_ref, lb_ref, b_ref, wc1_ref, bc1_ref, wc2_ref,
               bc2_ref, wc3_ref, bc3_ref, o_ref):
    hn = jnp.maximum(_ln(h_ref[...], ls_ref[...], lb_ref[...]), 0.0)
    bvec = b_ref[0, :]
    gids = lax.broadcasted_iota(jnp.int32, (G, N), 0)
    oh = (gids == bvec[None, :]).astype(f32)
    summ = jnp.dot(oh, hn, preferred_element_type=f32)
    cnt = jnp.sum(oh, axis=1, keepdims=True)
    pooled = summ / jnp.maximum(cnt, 1.0)
    z = jnp.maximum(
        jnp.dot(pooled, wc1_ref[...], preferred_element_type=f32) + bc1_ref[...],
        0.0)
    z = jnp.maximum(
        jnp.dot(z, wc2_ref[...], preferred_element_type=f32) + bc2_ref[...],
        0.0)
    z3 = jnp.dot(z, wc3_ref[...], preferred_element_type=f32) + bc3_ref[...]
    o_ref[...] = z3


def _head(h, ls, lb, batch2d, Wc1, bc1, Wc2, bc2, Wc3b, bc3b):
    return pl.pallas_call(
        _head_body,
        out_shape=jax.ShapeDtypeStruct((G, H), f32),
    )(h, ls, lb, batch2d, Wc1, bc1, Wc2, bc2, Wc3b, bc3b)


def _edge_body(r_hbm, q_hbm, src_hbm, dst_hbm, zero_hbm, numer_hbm, denom_hbm,
               src_v, dst_v, rows_v, acc):
    c = lax.axis_index("c")
    s = lax.axis_index("s")

    pltpu.sync_copy(zero_hbm, acc.at[pl.ds(s * ZR, ZR)])
    plsc.subcore_barrier()

    base = s * EPT

    @pl.loop(0, NCH_PER_TILE)
    def _chunk(k):
        e0 = base + k * CH
        pltpu.sync_copy(src_hbm.at[pl.ds(e0, CH)], src_v)
        pltpu.sync_copy(dst_hbm.at[pl.ds(e0, CH)], dst_v)

        @pl.when(c == 0)
        def _():
            pltpu.sync_copy(r_hbm.at[src_v], rows_v)

        @pl.when(c == 1)
        def _():
            pltpu.sync_copy(q_hbm.at[src_v], rows_v)

        pltpu.sync_copy(rows_v, acc.at[dst_v], add=True)

    plsc.subcore_barrier()

    @pl.when(c == 0)
    def _():
        pltpu.sync_copy(acc.at[pl.ds(s * ZR, ZR)],
                        numer_hbm.at[pl.ds(s * ZR, ZR)])

    @pl.when(c == 1)
    def _():
        pltpu.sync_copy(acc.at[pl.ds(s * ZR, ZR)],
                        denom_hbm.at[pl.ds(s * ZR, ZR)])


@functools.lru_cache(maxsize=1)
def _edge_call_built():
    return pl.kernel(
        _edge_body,
        out_type=[jax.ShapeDtypeStruct((NPAD, H), f32),
                  jax.ShapeDtypeStruct((NPAD, H), f32)],
        mesh=plsc.VectorSubcoreMesh(core_axis_name="c", subcore_axis_name="s"),
        scratch_types=[
            pltpu.VMEM((CH,), jnp.int32),
            pltpu.VMEM((CH,), jnp.int32),
            pltpu.VMEM((CH, H), f32),
            pltpu.VMEM_SHARED((NPAD, H), f32),
        ],
    )


def _edge_call(r, q, srcp, dstp, zeros):
    return _edge_call_built()(r, q, srcp, dstp, zeros)


def kernel(x, edge_index, batch, W_in, b_in, ln_scale, ln_bias, t, W1, b1,
           mlp_ln_scale, mlp_ln_bias, W2, b2, Wc1, bc1, Wc2, bc2, Wc3, bc3):
    src = edge_index[0]
    dst = edge_index[1]
    pad = EPAD - E
    srcp = jnp.concatenate([src, jnp.zeros((pad,), jnp.int32)])
    dstp = jnp.concatenate([dst, jnp.full((pad,), NPAD - 8, jnp.int32)])
    zeros = jnp.zeros((ZR, H), f32)

    h = _input_proj(x, W_in, b_in)
    for i in range(L):
        tv = jnp.full((1, H), t[i], f32)
        hn, r, q = _pre(h, ln_scale[i].reshape(1, H), ln_bias[i].reshape(1, H),
                        tv)
        numer, denom = _edge_call(r, q, srcp, dstp, zeros)
        h = _post(h, hn, numer[:N], denom[:N], W1[i], b1[i].reshape(1, H2),
                  mlp_ln_scale[i].reshape(1, H2), mlp_ln_bias[i].reshape(1, H2),
                  W2[i], b2[i].reshape(1, H))

    out2d = _head(
        h, ln_scale[0].reshape(1, H), ln_bias[0].reshape(1, H),
        batch.reshape(1, N), Wc1, bc1.reshape(1, H), Wc2,
        bc2.reshape(1, G), jnp.tile(Wc3, (1, H)),
        jnp.broadcast_to(bc3.reshape(1, 1), (1, H)))
    return out2d[:, 0]


# speedup vs baseline: 10.1387x; 1.2187x over previous
import functools

import jax
import jax.numpy as jnp
from jax import lax
from jax.experimental import pallas as pl
from jax.experimental.pallas import tpu as pltpu
from jax.experimental.pallas import tpu_sc as plsc

N = 10000
E = 320000
H = 128
H2 = 256
L = 7
G = 64

NSUB = 16
CH = 128
NCH_PER_TILE = 158
EPT = NCH_PER_TILE * CH
EPAD = EPT * NSUB
NPAD = 10112
ZR = NPAD // NSUB

f32 = jnp.float32


def _in_body(x_ref, w_ref, b_ref, o_ref):
    o_ref[...] = (
        jnp.dot(x_ref[...], w_ref[...], preferred_element_type=f32) + b_ref[...]
    )


def _input_proj(x, W_in, b_in):
    BM = 2000
    return pl.pallas_call(
        _in_body,
        grid=(N // BM,),
        in_specs=[
            pl.BlockSpec((BM, H), lambda i: (i, 0)),
            pl.BlockSpec((H, H), lambda i: (0, 0)),
            pl.BlockSpec((1, H), lambda i: (0, 0)),
        ],
        out_specs=pl.BlockSpec((BM, H), lambda i: (i, 0)),
        out_shape=jax.ShapeDtypeStruct((N, H), f32),
    )(x, W_in, b_in.reshape(1, H))


def _ln(h, scale, bias):
    mu = jnp.mean(h, axis=1, keepdims=True)
    var = jnp.mean((h - mu) ** 2, axis=1, keepdims=True)
    return (h - mu) / jnp.sqrt(var + 1e-5) * scale + bias


def _pre_body(h_ref, ls_ref, lb_ref, tv_ref, hn_ref, r_ref, q_ref):
    hn = jnp.maximum(_ln(h_ref[...], ls_ref[...], lb_ref[...]), 0.0)
    msg = hn + 1e-7
    q = jnp.exp(msg * tv_ref[...])
    hn_ref[...] = hn
    q_ref[...] = q
    r_ref[...] = msg * q


def _pre(h, ls, lb, tv):
    BM = 2000
    return pl.pallas_call(
        _pre_body,
        grid=(N // BM,),
        in_specs=[
            pl.BlockSpec((BM, H), lambda i: (i, 0)),
            pl.BlockSpec((1, H), lambda i: (0, 0)),
            pl.BlockSpec((1, H), lambda i: (0, 0)),
            pl.BlockSpec((1, H), lambda i: (0, 0)),
        ],
        out_specs=[pl.BlockSpec((BM, H), lambda i: (i, 0))] * 3,
        out_shape=[jax.ShapeDtypeStruct((N, H), f32)] * 3,
    )(h, ls, lb, tv)


def _post_body(h_ref, hn_ref, nu_ref, de_ref, w1_ref, b1_ref, mls_ref,
               mlb_ref, w2_ref, b2_ref, o_ref):
    aggr = nu_ref[...] / (de_ref[...] + 1e-16)
    out0 = aggr + hn_ref[...]
    u = jnp.dot(out0, w1_ref[...], preferred_element_type=f32) + b1_ref[...]
    u = jnp.maximum(_ln(u, mls_ref[...], mlb_ref[...]), 0.0)
    v = jnp.dot(u, w2_ref[...], preferred_element_type=f32) + b2_ref[...]
    o_ref[...] = h_ref[...] + v


def _post(h, hn, numer, denom, W1, b1, mls, mlb, W2, b2):
    BM = 2000
    return pl.pallas_call(
        _post_body,
        grid=(N // BM,),
        in_specs=[
            pl.BlockSpec((BM, H), lambda i: (i, 0)),
            pl.BlockSpec((BM, H), lambda i: (i, 0)),
            pl.BlockSpec((BM, H), lambda i: (i, 0)),
            pl.BlockSpec((BM, H), lambda i: (i, 0)),
            pl.BlockSpec((H, H2), lambda i: (0, 0)),
            pl.BlockSpec((1, H2), lambda i: (0, 0)),
            pl.BlockSpec((1, H2), lambda i: (0, 0)),
            pl.BlockSpec((1, H2), lambda i: (0, 0)),
            pl.BlockSpec((H2, H), lambda i: (0, 0)),
            pl.BlockSpec((1, H), lambda i: (0, 0)),
        ],
        out_specs=pl.BlockSpec((BM, H), lambda i: (i, 0)),
        out_shape=jax.ShapeDtypeStruct((N, H), f32),
    )(h, hn, numer, denom, W1, b1, mls, mlb, W2, b2)


def _head_body(h_ref, ls_ref, lb_ref, b_ref, wc1_ref, bc1_ref, wc2_ref,
               bc2_ref, wc3_ref, bc3_ref, o_ref):
    hn = jnp.maximum(_ln(h_ref[...], ls_ref[...], lb_ref[...]), 0.0)
    bvec = b_ref[0, :]
    gids = lax.broadcasted_iota(jnp.int32, (G, N), 0)
    oh = (gids == bvec[None, :]).astype(f32)
    summ = jnp.dot(oh, hn, preferred_element_type=f32)
    cnt = jnp.sum(oh, axis=1, keepdims=True)
    pooled = summ / jnp.maximum(cnt, 1.0)
    z = jnp.maximum(
        jnp.dot(pooled, wc1_ref[...], preferred_element_type=f32) + bc1_ref[...],
        0.0)
    z = jnp.maximum(
        jnp.dot(z, wc2_ref[...], preferred_element_type=f32) + bc2_ref[...],
        0.0)
    z3 = jnp.dot(z, wc3_ref[...], preferred_element_type=f32) + bc3_ref[...]
    o_ref[...] = z3


def _head(h, ls, lb, batch2d, Wc1, bc1, Wc2, bc2, Wc3b, bc3b):
    return pl.pallas_call(
        _head_body,
        out_shape=jax.ShapeDtypeStruct((G, H), f32),
    )(h, ls, lb, batch2d, Wc1, bc1, Wc2, bc2, Wc3b, bc3b)


def _edge_body(r_hbm, q_hbm, pk_hbm, zero_hbm, numer_hbm, denom_hbm,
               idx0, idx1, rows0, rows1, acc, i0, i1, g0, g1):
    c = lax.axis_index("c")
    s = lax.axis_index("s")

    pltpu.sync_copy(zero_hbm, acc.at[pl.ds(s * ZR, ZR)])
    plsc.subcore_barrier()

    def idx_start(k, idxb, isem):
        pltpu.async_copy(pk_hbm.at[s, k], idxb, isem)

    def idx_wait(idxb, isem):
        pltpu.make_async_copy(pk_hbm.at[s, 0], idxb, isem).wait()

    def gather_start(rows, idxb, gsem):
        @pl.when(c == 0)
        def _():
            pltpu.async_copy(r_hbm.at[idxb.at[0]], rows, gsem)

        @pl.when(c == 1)
        def _():
            pltpu.async_copy(q_hbm.at[idxb.at[0]], rows, gsem)

    def gather_wait(rows, idxb, gsem):
        pltpu.make_async_copy(r_hbm.at[idxb.at[0]], rows, gsem).wait()

    def scatter(rows, idxb):
        pltpu.sync_copy(rows, acc.at[idxb.at[1]], add=True)

    npair = NCH_PER_TILE // 2
    idx_start(0, idx0, i0)
    idx_start(1, idx1, i1)
    idx_wait(idx0, i0)
    gather_start(rows0, idx0, g0)

    @pl.loop(0, npair)
    def _pair(j):
        idx_wait(idx1, i1)
        gather_start(rows1, idx1, g1)
        gather_wait(rows0, idx0, g0)
        scatter(rows0, idx0)

        @pl.when(j + 1 < npair)
        def _():
            idx_start(2 * j + 2, idx0, i0)

        gather_wait(rows1, idx1, g1)
        scatter(rows1, idx1)

        @pl.when(j + 1 < npair)
        def _():
            idx_start(2 * j + 3, idx1, i1)
            idx_wait(idx0, i0)
            gather_start(rows0, idx0, g0)

    plsc.subcore_barrier()

    @pl.when(c == 0)
    def _():
        pltpu.sync_copy(acc.at[pl.ds(s * ZR, ZR)],
                        numer_hbm.at[pl.ds(s * ZR, ZR)])

    @pl.when(c == 1)
    def _():
        pltpu.sync_copy(acc.at[pl.ds(s * ZR, ZR)],
                        denom_hbm.at[pl.ds(s * ZR, ZR)])


@functools.lru_cache(maxsize=1)
def _edge_call_built():
    return pl.kernel(
        _edge_body,
        out_type=[jax.ShapeDtypeStruct((NPAD, H), f32),
                  jax.ShapeDtypeStruct((NPAD, H), f32)],
        mesh=plsc.VectorSubcoreMesh(core_axis_name="c", subcore_axis_name="s"),
        scratch_types=[
            pltpu.VMEM((2, CH), jnp.int32),
            pltpu.VMEM((2, CH), jnp.int32),
            pltpu.VMEM((CH, H), f32),
            pltpu.VMEM((CH, H), f32),
            pltpu.VMEM_SHARED((NPAD, H), f32),
            pltpu.SemaphoreType.DMA,
            pltpu.SemaphoreType.DMA,
            pltpu.SemaphoreType.DMA,
            pltpu.SemaphoreType.DMA,
        ],
    )


def _edge_call(r, q, srcp, dstp, zeros):
    pk = jnp.stack([srcp.reshape(NSUB, NCH_PER_TILE, CH),
                    dstp.reshape(NSUB, NCH_PER_TILE, CH)], axis=2)
    return _edge_call_built()(r, q, pk, zeros)


def kernel(x, edge_index, batch, W_in, b_in, ln_scale, ln_bias, t, W1, b1,
           mlp_ln_scale, mlp_ln_bias, W2, b2, Wc1, bc1, Wc2, bc2, Wc3, bc3):
    src = edge_index[0]
    dst = edge_index[1]
    pad = EPAD - E
    srcp = jnp.concatenate([src, jnp.zeros((pad,), jnp.int32)])
    dstp = jnp.concatenate([dst, jnp.full((pad,), NPAD - 8, jnp.int32)])
    zeros = jnp.zeros((ZR, H), f32)

    h = _input_proj(x, W_in, b_in)
    for i in range(L):
        tv = jnp.full((1, H), t[i], f32)
        hn, r, q = _pre(h, ln_scale[i].reshape(1, H), ln_bias[i].reshape(1, H),
                        tv)
        numer, denom = _edge_call(r, q, srcp, dstp, zeros)
        h = _post(h, hn, numer[:N], denom[:N], W1[i], b1[i].reshape(1, H2),
                  mlp_ln_scale[i].reshape(1, H2), mlp_ln_bias[i].reshape(1, H2),
                  W2[i], b2[i].reshape(1, H))

    out2d = _head(
        h, ln_scale[0].reshape(1, H), ln_bias[0].reshape(1, H),
        batch.reshape(1, N), Wc1, bc1.reshape(1, H), Wc2,
        bc2.reshape(1, G), jnp.tile(Wc3, (1, H)),
        jnp.broadcast_to(bc3.reshape(1, 1), (1, H)))
    return out2d[:, 0]
